# baseline (speedup 1.0000x reference)
import functools

import jax
import jax.numpy as jnp
from jax import lax
from jax.experimental import pallas as pl
from jax.experimental.pallas import tpu as pltpu
from jax.experimental.pallas import tpu_sc as plsc

T, N, D, H = 8, 10000, 128, 16
NPAD = 10240
NW = 32
ROWS_PER_TILE = NPAD // 16
F = T * H

KA = 8
KB = 2
NB = 1024

_MESH = dict(core_axis_name="c", subcore_axis_name="s",
             num_cores=2, num_subcores=16)


def _deg_body(n_chunks, dst_hbm, ones_hbm, zeros_hbm, out_hbm,
              idx_v, ones_v, deg_sh):
    c = lax.axis_index("c")
    s = lax.axis_index("s")
    wid = c * 16 + s
    r0 = s * ROWS_PER_TILE
    pltpu.sync_copy(zeros_hbm.at[pl.ds(0, 128)], ones_v)
    for z in range(ROWS_PER_TILE // 128):
        pltpu.sync_copy(ones_v, deg_sh.at[pl.ds(r0 + z * 128, 128)])
    pltpu.sync_copy(ones_hbm, ones_v)
    plsc.subcore_barrier()
    row_base = wid * (n_chunks * KA)

    def chunk(it, carry):
        pltpu.sync_copy(dst_hbm.at[pl.ds(row_base + it * KA, KA)], idx_v)
        for j in range(KA):
            pltpu.sync_copy(ones_v, deg_sh.at[idx_v.at[j]], add=True)
        return carry

    lax.fori_loop(0, n_chunks, chunk, 0)
    plsc.subcore_barrier()
    for z in range(ROWS_PER_TILE // 128):
        pltpu.sync_copy(deg_sh.at[pl.ds(r0 + z * 128, 128)], ones_v)
        pltpu.sync_copy(ones_v, out_hbm.at[c, pl.ds(r0 + z * 128, 128)])


def _agg_body(n_batches, hs_hbm, src_hbm, dst_hbm, zeros_hbm, out_hbm,
              sidx_v, didx_v, rows_v, acc_sh,
              sem_a, sem_b, sem_c, sem_d):
    c = lax.axis_index("c")
    s = lax.axis_index("s")
    wid = c * 16 + s
    r0 = s * ROWS_PER_TILE
    pltpu.sync_copy(zeros_hbm.at[pl.ds(0, 256)], rows_v)
    for z, zn in ((0, 256), (256, 256), (512, 128)):
        pltpu.sync_copy(rows_v.at[pl.ds(0, zn)],
                        acc_sh.at[pl.ds(r0 + z, zn)])
    plsc.subcore_barrier()

    def batch(b, carry):
        row0 = wid * (n_batches * 8) + b * KB
        pltpu.sync_copy(src_hbm.at[pl.ds(row0, KB)], sidx_v)
        pltpu.sync_copy(dst_hbm.at[pl.ds(row0, KB)], didx_v)
        descs = [
            pltpu.async_copy(hs_hbm.at[sidx_v.at[j]],
                             rows_v.at[pl.ds(j * 128, 128)], sem_a)
            for j in range(KB)
        ]
        for d in descs:
            d.wait()
        for j in range(KB):
            pltpu.sync_copy(rows_v.at[pl.ds(j * 128, 128)],
                            acc_sh.at[didx_v.at[j]], add=True)
        return carry

    lax.fori_loop(0, n_batches * 4, batch, 0)
    plsc.subcore_barrier()
    for z, zn in ((0, 256), (256, 256), (512, 128)):
        pltpu.sync_copy(acc_sh.at[pl.ds(r0 + z, zn)],
                        rows_v.at[pl.ds(0, zn)])
        pltpu.sync_copy(rows_v.at[pl.ds(0, zn)],
                        out_hbm.at[c, pl.ds(r0 + z, zn)])


def _hs_body(x_ref, degs_ref, w_ref, hs_ref, dis_ref):
    i = pl.program_id(0)
    deg = degs_ref[0, :, 0:1] + degs_ref[1, :, 0:1] + 1.0
    dis = lax.rsqrt(deg)
    rows = i * NB + lax.broadcasted_iota(jnp.int32, (NB, 1), 0)
    valid = rows < N
    parts = [jnp.dot(x_ref[t], w_ref[...], preferred_element_type=jnp.float32)
             for t in range(T)]
    h = jnp.concatenate(parts, axis=1)
    hs_ref[...] = jnp.where(valid, h * dis, 0.0)
    dis_ref[...] = jnp.where(valid, jnp.broadcast_to(dis, (NB, H)), 0.0)


def _lstm_body(acc_ref, hs_ref, dis_ref, bias_ref, wih_ref, whh_ref,
               b_ref, fcw_ref, fcb_ref, out_ref):
    dis = dis_ref[...][:, 0:1]
    agg = dis * (acc_ref[0] + acc_ref[1] + hs_ref[...])
    g = jnp.maximum(agg + bias_ref[...], 0.0)
    h = jnp.zeros((NB, H), jnp.float32)
    cst = jnp.zeros((NB, H), jnp.float32)
    b = b_ref[...]
    for t in range(T):
        xt = g[:, t * H:(t + 1) * H]
        gates = (jnp.dot(xt, wih_ref[...], preferred_element_type=jnp.float32)
                 + jnp.dot(h, whh_ref[...], preferred_element_type=jnp.float32)
                 + b)
        ii = jax.nn.sigmoid(gates[:, 0:H])
        ff = jax.nn.sigmoid(gates[:, H:2 * H])
        gg = jnp.tanh(gates[:, 2 * H:3 * H])
        oo = jax.nn.sigmoid(gates[:, 3 * H:4 * H])
        cst = ff * cst + ii * gg
        h = oo * jnp.tanh(cst)
    out_ref[...] = (jnp.dot(h, fcw_ref[...], preferred_element_type=jnp.float32)
                    + fcb_ref[...])


def kernel(x_list, edge_index, gcn_weight, gcn_bias, W_ih, W_hh, b_ih, b_hh,
           fc_w, fc_b):
    e = edge_index.shape[1]
    ept = KA * KB * 128
    e_pad = ((e + NW * ept - 1) // (NW * ept)) * (NW * ept)
    n_chunks_a = e_pad // (NW * KA * 128)
    n_batches = e_pad // (NW * 8 * 128)

    pad = N + (jnp.arange(e_pad - e, dtype=jnp.int32) % (NPAD - N))
    src2d = jnp.concatenate([edge_index[0], pad]).reshape(-1, 128)
    dst2d = jnp.concatenate([edge_index[1], pad]).reshape(-1, 128)

    ones128 = jnp.ones((128, F), jnp.float32)
    zerosF = jnp.zeros((NPAD, F), jnp.float32)

    deg_k = pl.kernel(
        functools.partial(_deg_body, n_chunks_a),
        out_type=jax.ShapeDtypeStruct((2, NPAD, F), jnp.float32),
        mesh=plsc.VectorSubcoreMesh(**_MESH),
        scratch_types=[
            pltpu.VMEM((KA, 128), jnp.int32),
            pltpu.VMEM((128, F), jnp.float32),
            pltpu.VMEM_SHARED((NPAD, F), jnp.float32),
        ],
    )
    degs = deg_k(dst2d, ones128, zerosF)

    grid = NPAD // NB
    hs, dis16 = pl.pallas_call(
        _hs_body,
        grid=(grid,),
        in_specs=[
            pl.BlockSpec((T, NB, D), lambda i: (0, i, 0)),
            pl.BlockSpec((2, NB, F), lambda i: (0, i, 0)),
            pl.BlockSpec((D, H), lambda i: (0, 0)),
        ],
        out_specs=[
            pl.BlockSpec((NB, F), lambda i: (i, 0)),
            pl.BlockSpec((NB, H), lambda i: (i, 0)),
        ],
        out_shape=[
            jax.ShapeDtypeStruct((NPAD, F), jnp.float32),
            jax.ShapeDtypeStruct((NPAD, H), jnp.float32),
        ],
    )(x_list, degs, gcn_weight)

    agg_k = pl.kernel(
        functools.partial(_agg_body, n_batches),
        out_type=jax.ShapeDtypeStruct((2, NPAD, F), jnp.float32),
        mesh=plsc.VectorSubcoreMesh(**_MESH),
        scratch_types=[
            pltpu.VMEM((KB, 128), jnp.int32),
            pltpu.VMEM((KB, 128), jnp.int32),
            pltpu.VMEM((KB * 128, F), jnp.float32),
            pltpu.VMEM_SHARED((NPAD, F), jnp.float32),
            pltpu.SemaphoreType.DMA,
            pltpu.SemaphoreType.DMA,
            pltpu.SemaphoreType.DMA,
            pltpu.SemaphoreType.DMA,
        ],
    )
    acc = agg_k(hs, src2d, dst2d, zerosF)

    bias128 = jnp.tile(gcn_bias, T).reshape(1, F)
    b_all = (b_ih + b_hh).reshape(1, 4 * H)
    out = pl.pallas_call(
        _lstm_body,
        grid=(grid,),
        in_specs=[
            pl.BlockSpec((2, NB, F), lambda i: (0, i, 0)),
            pl.BlockSpec((NB, F), lambda i: (i, 0)),
            pl.BlockSpec((NB, H), lambda i: (i, 0)),
            pl.BlockSpec((1, F), lambda i: (0, 0)),
            pl.BlockSpec((H, 4 * H), lambda i: (0, 0)),
            pl.BlockSpec((H, 4 * H), lambda i: (0, 0)),
            pl.BlockSpec((1, 4 * H), lambda i: (0, 0)),
            pl.BlockSpec((H, 1), lambda i: (0, 0)),
            pl.BlockSpec((1, 1), lambda i: (0, 0)),
        ],
        out_specs=pl.BlockSpec((NB, 1), lambda i: (i, 0)),
        out_shape=jax.ShapeDtypeStruct((NPAD, 1), jnp.float32),
    )(acc, hs, dis16, bias128, W_ih.T, W_hh.T, b_all, fc_w.T,
      fc_b.reshape(1, 1))

    return out[:N, 0]


# speedup vs baseline: 1.1762x; 1.1762x over previous
import functools

import jax
import jax.numpy as jnp
from jax import lax
from jax.experimental import pallas as pl
from jax.experimental.pallas import tpu as pltpu
from jax.experimental.pallas import tpu_sc as plsc

T, N, D, H = 8, 10000, 128, 16
NPAD = 10240
NW = 32
ROWS_PER_TILE = NPAD // 16
F = T * H

KA = 8
KB = 2
NB = 1024

_MESH = dict(core_axis_name="c", subcore_axis_name="s",
             num_cores=2, num_subcores=16)


def _deg_body(n_chunks, dst_hbm, ones_hbm, zeros_hbm, out_hbm,
              idx_v, ones_v, deg_sh):
    c = lax.axis_index("c")
    s = lax.axis_index("s")
    wid = c * 16 + s
    r0 = s * ROWS_PER_TILE
    pltpu.sync_copy(zeros_hbm.at[pl.ds(0, 128)], ones_v)
    for z in range(ROWS_PER_TILE // 128):
        pltpu.sync_copy(ones_v, deg_sh.at[pl.ds(r0 + z * 128, 128)])
    pltpu.sync_copy(ones_hbm, ones_v)
    plsc.subcore_barrier()
    row_base = wid * (n_chunks * KA)

    def chunk(it, carry):
        pltpu.sync_copy(dst_hbm.at[pl.ds(row_base + it * KA, KA)], idx_v)
        for j in range(KA):
            pltpu.sync_copy(ones_v, deg_sh.at[idx_v.at[j]], add=True)
        return carry

    lax.fori_loop(0, n_chunks, chunk, 0)
    plsc.subcore_barrier()
    for z in range(ROWS_PER_TILE // 128):
        pltpu.sync_copy(deg_sh.at[pl.ds(r0 + z * 128, 128)], ones_v)
        pltpu.sync_copy(ones_v, out_hbm.at[c, pl.ds(r0 + z * 128, 128)])


def _agg_body(n_batches, hs_hbm, src_hbm, dst_hbm, zeros_hbm, out_hbm,
              sidx_v, didx_v, buf_a, buf_b, acc_sh, sem_a, sem_b):
    c = lax.axis_index("c")
    s = lax.axis_index("s")
    wid = c * 16 + s
    r0 = s * ROWS_PER_TILE
    pltpu.sync_copy(zeros_hbm.at[pl.ds(0, 128)], buf_a)
    for z in range(ROWS_PER_TILE // 128):
        pltpu.sync_copy(buf_a, acc_sh.at[pl.ds(r0 + z * 128, 128)])
    plsc.subcore_barrier()

    bufs = (buf_a, buf_b)
    sems = (sem_a, sem_b)

    def gather(j):
        return pltpu.async_copy(hs_hbm.at[sidx_v.at[j]], bufs[j % 2],
                                sems[j % 2])

    def drain(j):
        pltpu.make_async_copy(hs_hbm.at[sidx_v.at[j]], bufs[j % 2],
                              sems[j % 2]).wait()

    def scatter(j):
        pltpu.sync_copy(bufs[j % 2], acc_sh.at[didx_v.at[j]], add=True)

    def batch(b, carry):
        row0 = wid * (n_batches * 8) + b * 8
        pltpu.sync_copy(src_hbm.at[pl.ds(row0, 8)], sidx_v)
        pltpu.sync_copy(dst_hbm.at[pl.ds(row0, 8)], didx_v)
        gather(0)
        for j in range(1, 8):
            gather(j)
            drain(j - 1)
            scatter(j - 1)
        drain(7)
        scatter(7)
        return carry

    lax.fori_loop(0, n_batches, batch, 0)
    plsc.subcore_barrier()
    for z in range(ROWS_PER_TILE // 128):
        pltpu.sync_copy(acc_sh.at[pl.ds(r0 + z * 128, 128)], buf_a)
        pltpu.sync_copy(buf_a, out_hbm.at[c, pl.ds(r0 + z * 128, 128)])


def _hs_body(x_ref, degs_ref, w_ref, hs_ref, dis_ref):
    i = pl.program_id(0)
    deg = degs_ref[0, :, 0:1] + degs_ref[1, :, 0:1] + 1.0
    dis = lax.rsqrt(deg)
    rows = i * NB + lax.broadcasted_iota(jnp.int32, (NB, 1), 0)
    valid = rows < N
    parts = [jnp.dot(x_ref[t], w_ref[...], preferred_element_type=jnp.float32)
             for t in range(T)]
    h = jnp.concatenate(parts, axis=1)
    hs_ref[...] = jnp.where(valid, h * dis, 0.0)
    dis_ref[...] = jnp.where(valid, jnp.broadcast_to(dis, (NB, H)), 0.0)


def _lstm_body(acc_ref, hs_ref, dis_ref, bias_ref, wih_ref, whh_ref,
               b_ref, fcw_ref, fcb_ref, out_ref):
    dis = dis_ref[...][:, 0:1]
    agg = dis * (acc_ref[0] + acc_ref[1] + hs_ref[...])
    g = jnp.maximum(agg + bias_ref[...], 0.0)
    h = jnp.zeros((NB, H), jnp.float32)
    cst = jnp.zeros((NB, H), jnp.float32)
    b = b_ref[...]
    for t in range(T):
        xt = g[:, t * H:(t + 1) * H]
        gates = (jnp.dot(xt, wih_ref[...], preferred_element_type=jnp.float32)
                 + jnp.dot(h, whh_ref[...], preferred_element_type=jnp.float32)
                 + b)
        ii = jax.nn.sigmoid(gates[:, 0:H])
        ff = jax.nn.sigmoid(gates[:, H:2 * H])
        gg = jnp.tanh(gates[:, 2 * H:3 * H])
        oo = jax.nn.sigmoid(gates[:, 3 * H:4 * H])
        cst = ff * cst + ii * gg
        h = oo * jnp.tanh(cst)
    out_ref[...] = (jnp.dot(h, fcw_ref[...], preferred_element_type=jnp.float32)
                    + fcb_ref[...])


def kernel(x_list, edge_index, gcn_weight, gcn_bias, W_ih, W_hh, b_ih, b_hh,
           fc_w, fc_b):
    e = edge_index.shape[1]
    ept = KA * KB * 128
    e_pad = ((e + NW * ept - 1) // (NW * ept)) * (NW * ept)
    n_chunks_a = e_pad // (NW * KA * 128)
    n_batches = e_pad // (NW * 8 * 128)

    pad = N + (jnp.arange(e_pad - e, dtype=jnp.int32) % (NPAD - N))
    src2d = jnp.concatenate([edge_index[0], pad]).reshape(-1, 128)
    dst2d = jnp.concatenate([edge_index[1], pad]).reshape(-1, 128)

    ones128 = jnp.ones((128, F), jnp.float32)
    zerosF = jnp.zeros((NPAD, F), jnp.float32)

    deg_k = pl.kernel(
        functools.partial(_deg_body, n_chunks_a),
        out_type=jax.ShapeDtypeStruct((2, NPAD, F), jnp.float32),
        mesh=plsc.VectorSubcoreMesh(**_MESH),
        scratch_types=[
            pltpu.VMEM((KA, 128), jnp.int32),
            pltpu.VMEM((128, F), jnp.float32),
            pltpu.VMEM_SHARED((NPAD, F), jnp.float32),
        ],
    )
    degs = deg_k(dst2d, ones128, zerosF)

    grid = NPAD // NB
    hs, dis16 = pl.pallas_call(
        _hs_body,
        grid=(grid,),
        in_specs=[
            pl.BlockSpec((T, NB, D), lambda i: (0, i, 0)),
            pl.BlockSpec((2, NB, F), lambda i: (0, i, 0)),
            pl.BlockSpec((D, H), lambda i: (0, 0)),
        ],
        out_specs=[
            pl.BlockSpec((NB, F), lambda i: (i, 0)),
            pl.BlockSpec((NB, H), lambda i: (i, 0)),
        ],
        out_shape=[
            jax.ShapeDtypeStruct((NPAD, F), jnp.float32),
            jax.ShapeDtypeStruct((NPAD, H), jnp.float32),
        ],
    )(x_list, degs, gcn_weight)

    agg_k = pl.kernel(
        functools.partial(_agg_body, n_batches),
        out_type=jax.ShapeDtypeStruct((2, NPAD, F), jnp.float32),
        mesh=plsc.VectorSubcoreMesh(**_MESH),
        scratch_types=[
            pltpu.VMEM((8, 128), jnp.int32),
            pltpu.VMEM((8, 128), jnp.int32),
            pltpu.VMEM((128, F), jnp.float32),
            pltpu.VMEM((128, F), jnp.float32),
            pltpu.VMEM_SHARED((NPAD, F), jnp.float32),
            pltpu.SemaphoreType.DMA,
            pltpu.SemaphoreType.DMA,
        ],
    )
    acc = agg_k(hs, src2d, dst2d, zerosF)

    bias128 = jnp.tile(gcn_bias, T).reshape(1, F)
    b_all = (b_ih + b_hh).reshape(1, 4 * H)
    out = pl.pallas_call(
        _lstm_body,
        grid=(grid,),
        in_specs=[
            pl.BlockSpec((2, NB, F), lambda i: (0, i, 0)),
            pl.BlockSpec((NB, F), lambda i: (i, 0)),
            pl.BlockSpec((NB, H), lambda i: (i, 0)),
            pl.BlockSpec((1, F), lambda i: (0, 0)),
            pl.BlockSpec((H, 4 * H), lambda i: (0, 0)),
            pl.BlockSpec((H, 4 * H), lambda i: (0, 0)),
            pl.BlockSpec((1, 4 * H), lambda i: (0, 0)),
            pl.BlockSpec((H, 1), lambda i: (0, 0)),
            pl.BlockSpec((1, 1), lambda i: (0, 0)),
        ],
        out_specs=pl.BlockSpec((NB, 1), lambda i: (i, 0)),
        out_shape=jax.ShapeDtypeStruct((NPAD, 1), jnp.float32),
    )(acc, hs, dis16, bias128, W_ih.T, W_hh.T, b_all, fc_w.T,
      fc_b.reshape(1, 1))

    return out[:N, 0]


# speedup vs baseline: 1.2075x; 1.0265x over previous
import functools

import jax
import jax.numpy as jnp
from jax import lax
from jax.experimental import pallas as pl
from jax.experimental.pallas import tpu as pltpu
from jax.experimental.pallas import tpu_sc as plsc

T, N, D, H = 8, 10000, 128, 16
NPAD = 10240
NW = 32
ROWS_PER_TILE = NPAD // 16
F = T * H

KA = 8
KB = 2
NB = 1024

_MESH = dict(core_axis_name="c", subcore_axis_name="s",
             num_cores=2, num_subcores=16)


def _deg_body(n_chunks, dst_hbm, ones_hbm, zeros_hbm, out_hbm,
              idx_v, ones_v, deg_sh, sem_a, sem_b):
    c = lax.axis_index("c")
    s = lax.axis_index("s")
    wid = c * 16 + s
    r0 = s * ROWS_PER_TILE
    pltpu.sync_copy(zeros_hbm.at[pl.ds(0, 128)], ones_v)
    for z in range(ROWS_PER_TILE // 128):
        pltpu.sync_copy(ones_v, deg_sh.at[pl.ds(r0 + z * 128, 128)])
    pltpu.sync_copy(ones_hbm, ones_v)
    plsc.subcore_barrier()
    row_base = wid * (n_chunks * KA)

    def chunk(it, carry):
        pltpu.sync_copy(dst_hbm.at[pl.ds(row_base + it * KA, KA)], idx_v)
        descs = [
            pltpu.async_copy(ones_v, deg_sh.at[idx_v.at[j]],
                             (sem_a, sem_b)[j % 2], add=True)
            for j in range(KA)
        ]
        for d in descs:
            d.wait()
        return carry

    lax.fori_loop(0, n_chunks, chunk, 0)
    plsc.subcore_barrier()
    for z in range(ROWS_PER_TILE // 128):
        pltpu.sync_copy(deg_sh.at[pl.ds(r0 + z * 128, 128)], ones_v)
        pltpu.sync_copy(ones_v, out_hbm.at[c, pl.ds(r0 + z * 128, 128)])


def _agg_body(n_batches, hs_hbm, src_hbm, dst_hbm, zeros_hbm, out_hbm,
              sidx_v, didx_v, buf_a, buf_b, acc_sh, sem_a, sem_b):
    c = lax.axis_index("c")
    s = lax.axis_index("s")
    wid = c * 16 + s
    r0 = s * ROWS_PER_TILE
    pltpu.sync_copy(zeros_hbm.at[pl.ds(0, 128)], buf_a)
    for z in range(ROWS_PER_TILE // 128):
        pltpu.sync_copy(buf_a, acc_sh.at[pl.ds(r0 + z * 128, 128)])
    plsc.subcore_barrier()

    bufs = (buf_a, buf_b)
    sems = (sem_a, sem_b)

    def gather(j):
        return pltpu.async_copy(hs_hbm.at[sidx_v.at[j]], bufs[j % 2],
                                sems[j % 2])

    def drain(j):
        pltpu.make_async_copy(hs_hbm.at[sidx_v.at[j]], bufs[j % 2],
                              sems[j % 2]).wait()

    def scatter(j):
        pltpu.sync_copy(bufs[j % 2], acc_sh.at[didx_v.at[j]], add=True)

    def batch(b, carry):
        row0 = wid * (n_batches * 8) + b * 8
        pltpu.sync_copy(src_hbm.at[pl.ds(row0, 8)], sidx_v)
        pltpu.sync_copy(dst_hbm.at[pl.ds(row0, 8)], didx_v)
        gather(0)
        for j in range(1, 8):
            gather(j)
            drain(j - 1)
            scatter(j - 1)
        drain(7)
        scatter(7)
        return carry

    lax.fori_loop(0, n_batches, batch, 0)
    plsc.subcore_barrier()
    for z in range(ROWS_PER_TILE // 128):
        pltpu.sync_copy(acc_sh.at[pl.ds(r0 + z * 128, 128)], buf_a)
        pltpu.sync_copy(buf_a, out_hbm.at[c, pl.ds(r0 + z * 128, 128)])


def _h_body(x_ref, w_ref, h_ref):
    parts = [jnp.dot(x_ref[t], w_ref[...], preferred_element_type=jnp.float32)
             for t in range(T)]
    h_ref[...] = jnp.concatenate(parts, axis=1)


def _scale_body(h_ref, degs_ref, hs_ref, dis_ref):
    i = pl.program_id(0)
    deg = degs_ref[0, :, 0:1] + degs_ref[1, :, 0:1] + 1.0
    dis = lax.rsqrt(deg)
    rows = i * NB + lax.broadcasted_iota(jnp.int32, (NB, 1), 0)
    valid = rows < N
    hs_ref[...] = jnp.where(valid, h_ref[...] * dis, 0.0)
    dis_ref[...] = jnp.where(valid, jnp.broadcast_to(dis, (NB, H)), 0.0)


def _lstm_body(acc_ref, hs_ref, dis_ref, bias_ref, wih_ref, whh_ref,
               b_ref, fcw_ref, fcb_ref, out_ref):
    dis = dis_ref[...][:, 0:1]
    agg = dis * (acc_ref[0] + acc_ref[1] + hs_ref[...])
    g = jnp.maximum(agg + bias_ref[...], 0.0)
    h = jnp.zeros((NB, H), jnp.float32)
    cst = jnp.zeros((NB, H), jnp.float32)
    b = b_ref[...]
    for t in range(T):
        xt = g[:, t * H:(t + 1) * H]
        gates = (jnp.dot(xt, wih_ref[...], preferred_element_type=jnp.float32)
                 + jnp.dot(h, whh_ref[...], preferred_element_type=jnp.float32)
                 + b)
        ii = jax.nn.sigmoid(gates[:, 0:H])
        ff = jax.nn.sigmoid(gates[:, H:2 * H])
        gg = jnp.tanh(gates[:, 2 * H:3 * H])
        oo = jax.nn.sigmoid(gates[:, 3 * H:4 * H])
        cst = ff * cst + ii * gg
        h = oo * jnp.tanh(cst)
    out_ref[...] = (jnp.dot(h, fcw_ref[...], preferred_element_type=jnp.float32)
                    + fcb_ref[...])


def kernel(x_list, edge_index, gcn_weight, gcn_bias, W_ih, W_hh, b_ih, b_hh,
           fc_w, fc_b):
    e = edge_index.shape[1]
    ept = KA * KB * 128
    e_pad = ((e + NW * ept - 1) // (NW * ept)) * (NW * ept)
    n_chunks_a = e_pad // (NW * KA * 128)
    n_batches = e_pad // (NW * 8 * 128)

    pad = N + (jnp.arange(e_pad - e, dtype=jnp.int32) % (NPAD - N))
    src2d = jnp.concatenate([edge_index[0], pad]).reshape(-1, 128)
    dst2d = jnp.concatenate([edge_index[1], pad]).reshape(-1, 128)

    ones128 = jnp.ones((128, F), jnp.float32)
    zerosF = jnp.zeros((NPAD, F), jnp.float32)

    deg_k = pl.kernel(
        functools.partial(_deg_body, n_chunks_a),
        out_type=jax.ShapeDtypeStruct((2, NPAD, F), jnp.float32),
        mesh=plsc.VectorSubcoreMesh(**_MESH),
        scratch_types=[
            pltpu.VMEM((KA, 128), jnp.int32),
            pltpu.VMEM((128, F), jnp.float32),
            pltpu.VMEM_SHARED((NPAD, F), jnp.float32),
            pltpu.SemaphoreType.DMA,
            pltpu.SemaphoreType.DMA,
        ],
    )
    degs = deg_k(dst2d, ones128, zerosF)

    grid = NPAD // NB
    h_all = pl.pallas_call(
        _h_body,
        grid=(grid,),
        in_specs=[
            pl.BlockSpec((T, NB, D), lambda i: (0, i, 0)),
            pl.BlockSpec((D, H), lambda i: (0, 0)),
        ],
        out_specs=pl.BlockSpec((NB, F), lambda i: (i, 0)),
        out_shape=jax.ShapeDtypeStruct((NPAD, F), jnp.float32),
    )(x_list, gcn_weight)
    hs, dis16 = pl.pallas_call(
        _scale_body,
        grid=(grid,),
        in_specs=[
            pl.BlockSpec((NB, F), lambda i: (i, 0)),
            pl.BlockSpec((2, NB, F), lambda i: (0, i, 0)),
        ],
        out_specs=[
            pl.BlockSpec((NB, F), lambda i: (i, 0)),
            pl.BlockSpec((NB, H), lambda i: (i, 0)),
        ],
        out_shape=[
            jax.ShapeDtypeStruct((NPAD, F), jnp.float32),
            jax.ShapeDtypeStruct((NPAD, H), jnp.float32),
        ],
    )(h_all, degs)

    agg_k = pl.kernel(
        functools.partial(_agg_body, n_batches),
        out_type=jax.ShapeDtypeStruct((2, NPAD, F), jnp.float32),
        mesh=plsc.VectorSubcoreMesh(**_MESH),
        scratch_types=[
            pltpu.VMEM((8, 128), jnp.int32),
            pltpu.VMEM((8, 128), jnp.int32),
            pltpu.VMEM((128, F), jnp.float32),
            pltpu.VMEM((128, F), jnp.float32),
            pltpu.VMEM_SHARED((NPAD, F), jnp.float32),
            pltpu.SemaphoreType.DMA,
            pltpu.SemaphoreType.DMA,
        ],
    )
    acc = agg_k(hs, src2d, dst2d, zerosF)

    bias128 = jnp.tile(gcn_bias, T).reshape(1, F)
    b_all = (b_ih + b_hh).reshape(1, 4 * H)
    out = pl.pallas_call(
        _lstm_body,
        grid=(grid,),
        in_specs=[
            pl.BlockSpec((2, NB, F), lambda i: (0, i, 0)),
            pl.BlockSpec((NB, F), lambda i: (i, 0)),
            pl.BlockSpec((NB, H), lambda i: (i, 0)),
            pl.BlockSpec((1, F), lambda i: (0, 0)),
            pl.BlockSpec((H, 4 * H), lambda i: (0, 0)),
            pl.BlockSpec((H, 4 * H), lambda i: (0, 0)),
            pl.BlockSpec((1, 4 * H), lambda i: (0, 0)),
            pl.BlockSpec((H, 1), lambda i: (0, 0)),
            pl.BlockSpec((1, 1), lambda i: (0, 0)),
        ],
        out_specs=pl.BlockSpec((NB, 1), lambda i: (i, 0)),
        out_shape=jax.ShapeDtypeStruct((NPAD, 1), jnp.float32),
    )(acc, hs, dis16, bias128, W_ih.T, W_hh.T, b_all, fc_w.T,
      fc_b.reshape(1, 1))

    return out[:N, 0]


# speedup vs baseline: 1.2415x; 1.0282x over previous
import functools

import jax
import jax.numpy as jnp
from jax import lax
from jax.experimental import pallas as pl
from jax.experimental.pallas import tpu as pltpu
from jax.experimental.pallas import tpu_sc as plsc

T, N, D, H = 8, 10000, 128, 16
NPAD = 10240
NW = 32
ROWS_PER_TILE = NPAD // 16
F = T * H

KA = 8
KB = 2
NB = 2048

_MESH = dict(core_axis_name="c", subcore_axis_name="s",
             num_cores=2, num_subcores=16)


def _deg_body(n_chunks, dst_hbm, ones_hbm, zeros_hbm, out_hbm,
              idx_v, ones_v, deg_sh, sem_a, sem_b):
    c = lax.axis_index("c")
    s = lax.axis_index("s")
    wid = c * 16 + s
    r0 = s * ROWS_PER_TILE
    pltpu.sync_copy(zeros_hbm.at[pl.ds(0, 128)], ones_v)
    for z in range(ROWS_PER_TILE // 128):
        pltpu.sync_copy(ones_v, deg_sh.at[pl.ds(r0 + z * 128, 128)])
    pltpu.sync_copy(ones_hbm, ones_v)
    plsc.subcore_barrier()
    row_base = wid * (n_chunks * KA)

    def chunk(it, carry):
        pltpu.sync_copy(dst_hbm.at[pl.ds(row_base + it * KA, KA)], idx_v)
        descs = [
            pltpu.async_copy(ones_v, deg_sh.at[idx_v.at[j]],
                             (sem_a, sem_b)[j % 2], add=True)
            for j in range(KA)
        ]
        for d in descs:
            d.wait()
        return carry

    lax.fori_loop(0, n_chunks, chunk, 0)
    plsc.subcore_barrier()
    for z in range(ROWS_PER_TILE // 128):
        pltpu.sync_copy(deg_sh.at[pl.ds(r0 + z * 128, 128)], ones_v)
        pltpu.sync_copy(ones_v, out_hbm.at[c, pl.ds(r0 + z * 128, 128)])


def _agg_body(n_batches, hs_hbm, src_hbm, dst_hbm, zeros_hbm, out_hbm,
              sidx_v, didx_v, buf_a, buf_b, acc_sh, sem_a, sem_b):
    c = lax.axis_index("c")
    s = lax.axis_index("s")
    wid = c * 16 + s
    r0 = s * ROWS_PER_TILE
    pltpu.sync_copy(zeros_hbm.at[pl.ds(0, 128)], buf_a)
    for z in range(ROWS_PER_TILE // 128):
        pltpu.sync_copy(buf_a, acc_sh.at[pl.ds(r0 + z * 128, 128)])
    plsc.subcore_barrier()

    bufs = (buf_a, buf_b)
    sems = (sem_a, sem_b)

    def gather(j):
        return pltpu.async_copy(hs_hbm.at[sidx_v.at[j]], bufs[j % 2],
                                sems[j % 2])

    def drain(j):
        pltpu.make_async_copy(hs_hbm.at[sidx_v.at[j]], bufs[j % 2],
                              sems[j % 2]).wait()

    def scatter(j):
        pltpu.sync_copy(bufs[j % 2], acc_sh.at[didx_v.at[j]], add=True)

    def batch(b, carry):
        row0 = wid * (n_batches * 8) + b * 8
        pltpu.sync_copy(src_hbm.at[pl.ds(row0, 8)], sidx_v)
        pltpu.sync_copy(dst_hbm.at[pl.ds(row0, 8)], didx_v)
        gather(0)
        for j in range(1, 8):
            gather(j)
            drain(j - 1)
            scatter(j - 1)
        drain(7)
        scatter(7)
        return carry

    lax.fori_loop(0, n_batches, batch, 0)
    plsc.subcore_barrier()
    for z in range(ROWS_PER_TILE // 128):
        pltpu.sync_copy(acc_sh.at[pl.ds(r0 + z * 128, 128)], buf_a)
        pltpu.sync_copy(buf_a, out_hbm.at[c, pl.ds(r0 + z * 128, 128)])


def _h_body(x_ref, w_ref, h_ref):
    parts = [jnp.dot(x_ref[t], w_ref[...], preferred_element_type=jnp.float32)
             for t in range(T)]
    h_ref[...] = jnp.concatenate(parts, axis=1)


def _scale_body(h_ref, degs_ref, hs_ref, dis_ref):
    i = pl.program_id(0)
    deg = degs_ref[0, :, 0:1] + degs_ref[1, :, 0:1] + 1.0
    dis = lax.rsqrt(deg)
    rows = i * NB + lax.broadcasted_iota(jnp.int32, (NB, 1), 0)
    valid = rows < N
    hs_ref[...] = jnp.where(valid, h_ref[...] * dis, 0.0)
    dis_ref[...] = jnp.where(valid, jnp.broadcast_to(dis, (NB, H)), 0.0)


def _lstm_body(acc_ref, hs_ref, dis_ref, bias_ref, wih_ref, whh_ref,
               b_ref, fcw_ref, fcb_ref, out_ref):
    dis = dis_ref[...][:, 0:1]
    agg = dis * (acc_ref[0] + acc_ref[1] + hs_ref[...])
    g = jnp.maximum(agg + bias_ref[...], 0.0)
    h = jnp.zeros((NB, H), jnp.float32)
    cst = jnp.zeros((NB, H), jnp.float32)
    b = b_ref[...]
    for t in range(T):
        xt = g[:, t * H:(t + 1) * H]
        gates = (jnp.dot(xt, wih_ref[...], preferred_element_type=jnp.float32)
                 + jnp.dot(h, whh_ref[...], preferred_element_type=jnp.float32)
                 + b)
        ii = jax.nn.sigmoid(gates[:, 0:H])
        ff = jax.nn.sigmoid(gates[:, H:2 * H])
        gg = jnp.tanh(gates[:, 2 * H:3 * H])
        oo = jax.nn.sigmoid(gates[:, 3 * H:4 * H])
        cst = ff * cst + ii * gg
        h = oo * jnp.tanh(cst)
    out_ref[...] = (jnp.dot(h, fcw_ref[...], preferred_element_type=jnp.float32)
                    + fcb_ref[...])


def kernel(x_list, edge_index, gcn_weight, gcn_bias, W_ih, W_hh, b_ih, b_hh,
           fc_w, fc_b):
    e = edge_index.shape[1]
    ept = KA * KB * 128
    e_pad = ((e + NW * ept - 1) // (NW * ept)) * (NW * ept)
    n_chunks_a = e_pad // (NW * KA * 128)
    n_batches = e_pad // (NW * 8 * 128)

    pad = N + (jnp.arange(e_pad - e, dtype=jnp.int32) % (NPAD - N))
    src2d = jnp.concatenate([edge_index[0], pad]).reshape(-1, 128)
    dst2d = jnp.concatenate([edge_index[1], pad]).reshape(-1, 128)

    ones128 = jnp.ones((128, F), jnp.float32)
    zerosF = jnp.zeros((NPAD, F), jnp.float32)

    deg_k = pl.kernel(
        functools.partial(_deg_body, n_chunks_a),
        out_type=jax.ShapeDtypeStruct((2, NPAD, F), jnp.float32),
        mesh=plsc.VectorSubcoreMesh(**_MESH),
        scratch_types=[
            pltpu.VMEM((KA, 128), jnp.int32),
            pltpu.VMEM((128, F), jnp.float32),
            pltpu.VMEM_SHARED((NPAD, F), jnp.float32),
            pltpu.SemaphoreType.DMA,
            pltpu.SemaphoreType.DMA,
        ],
    )
    degs = deg_k(dst2d, ones128, zerosF)

    grid = NPAD // NB
    h_all = pl.pallas_call(
        _h_body,
        grid=(grid,),
        in_specs=[
            pl.BlockSpec((T, NB, D), lambda i: (0, i, 0)),
            pl.BlockSpec((D, H), lambda i: (0, 0)),
        ],
        out_specs=pl.BlockSpec((NB, F), lambda i: (i, 0)),
        out_shape=jax.ShapeDtypeStruct((NPAD, F), jnp.float32),
    )(x_list, gcn_weight)
    hs, dis16 = pl.pallas_call(
        _scale_body,
        grid=(grid,),
        in_specs=[
            pl.BlockSpec((NB, F), lambda i: (i, 0)),
            pl.BlockSpec((2, NB, F), lambda i: (0, i, 0)),
        ],
        out_specs=[
            pl.BlockSpec((NB, F), lambda i: (i, 0)),
            pl.BlockSpec((NB, H), lambda i: (i, 0)),
        ],
        out_shape=[
            jax.ShapeDtypeStruct((NPAD, F), jnp.float32),
            jax.ShapeDtypeStruct((NPAD, H), jnp.float32),
        ],
    )(h_all, degs)

    agg_k = pl.kernel(
        functools.partial(_agg_body, n_batches),
        out_type=jax.ShapeDtypeStruct((2, NPAD, F), jnp.float32),
        mesh=plsc.VectorSubcoreMesh(**_MESH),
        scratch_types=[
            pltpu.VMEM((8, 128), jnp.int32),
            pltpu.VMEM((8, 128), jnp.int32),
            pltpu.VMEM((128, F), jnp.float32),
            pltpu.VMEM((128, F), jnp.float32),
            pltpu.VMEM_SHARED((NPAD, F), jnp.float32),
            pltpu.SemaphoreType.DMA,
            pltpu.SemaphoreType.DMA,
        ],
    )
    acc = agg_k(hs, src2d, dst2d, zerosF)

    bias128 = jnp.tile(gcn_bias, T).reshape(1, F)
    b_all = (b_ih + b_hh).reshape(1, 4 * H)
    out = pl.pallas_call(
        _lstm_body,
        grid=(grid,),
        in_specs=[
            pl.BlockSpec((2, NB, F), lambda i: (0, i, 0)),
            pl.BlockSpec((NB, F), lambda i: (i, 0)),
            pl.BlockSpec((NB, H), lambda i: (i, 0)),
            pl.BlockSpec((1, F), lambda i: (0, 0)),
            pl.BlockSpec((H, 4 * H), lambda i: (0, 0)),
            pl.BlockSpec((H, 4 * H), lambda i: (0, 0)),
            pl.BlockSpec((1, 4 * H), lambda i: (0, 0)),
            pl.BlockSpec((H, 1), lambda i: (0, 0)),
            pl.BlockSpec((1, 1), lambda i: (0, 0)),
        ],
        out_specs=pl.BlockSpec((NB, 1), lambda i: (i, 0)),
        out_shape=jax.ShapeDtypeStruct((NPAD, 1), jnp.float32),
    )(acc, hs, dis16, bias128, W_ih.T, W_hh.T, b_all, fc_w.T,
      fc_b.reshape(1, 1))

    return out[:N, 0]


# speedup vs baseline: 1.2867x; 1.0364x over previous
import functools

import jax
import jax.numpy as jnp
from jax import lax
from jax.experimental import pallas as pl
from jax.experimental.pallas import tpu as pltpu
from jax.experimental.pallas import tpu_sc as plsc

T, N, D, H = 8, 10000, 128, 16
NPAD = 10240
NW = 32
ROWS_PER_TILE = NPAD // 16
F = T * H

KA = 8
DEGW = 128
KB = 2
NB = 2048

_MESH = dict(core_axis_name="c", subcore_axis_name="s",
             num_cores=2, num_subcores=16)


def _deg_body(n_chunks, dst_hbm, ones_hbm, zeros_hbm, out_hbm,
              idx_v, ones_v, deg_sh, sem_a, sem_b):
    c = lax.axis_index("c")
    s = lax.axis_index("s")
    wid = c * 16 + s
    r0 = s * ROWS_PER_TILE
    pltpu.sync_copy(zeros_hbm.at[pl.ds(0, 128)], ones_v)
    for z in range(ROWS_PER_TILE // 128):
        pltpu.sync_copy(ones_v, deg_sh.at[pl.ds(r0 + z * 128, 128)])
    pltpu.sync_copy(ones_hbm, ones_v)
    plsc.subcore_barrier()
    row_base = wid * (n_chunks * KA)

    def chunk(it, carry):
        pltpu.sync_copy(dst_hbm.at[pl.ds(row_base + it * KA, KA)], idx_v)
        descs = [
            pltpu.async_copy(ones_v, deg_sh.at[idx_v.at[j]],
                             (sem_a, sem_b)[j % 2], add=True)
            for j in range(KA)
        ]
        for d in descs:
            d.wait()
        return carry

    lax.fori_loop(0, n_chunks, chunk, 0)
    plsc.subcore_barrier()
    for z in range(ROWS_PER_TILE // 128):
        pltpu.sync_copy(deg_sh.at[pl.ds(r0 + z * 128, 128)], ones_v)
        pltpu.sync_copy(ones_v, out_hbm.at[c, pl.ds(r0 + z * 128, 128)])


def _agg_body(n_batches, hs_hbm, src_hbm, dst_hbm, zeros_hbm, out_hbm,
              sidx_a, sidx_b, didx_a, didx_b, buf_a, buf_b, acc_sh,
              sem_a, sem_b, sem_i):
    c = lax.axis_index("c")
    s = lax.axis_index("s")
    wid = c * 16 + s
    r0 = s * ROWS_PER_TILE
    pltpu.sync_copy(zeros_hbm.at[pl.ds(0, 128)], buf_a)
    for z in range(ROWS_PER_TILE // 128):
        pltpu.sync_copy(buf_a, acc_sh.at[pl.ds(r0 + z * 128, 128)])
    plsc.subcore_barrier()

    bufs = (buf_a, buf_b)
    sems = (sem_a, sem_b)
    sidxs = (sidx_a, sidx_b)
    didxs = (didx_a, didx_b)
    row_base = wid * (n_batches * 8)
    last_row = row_base + (n_batches - 1) * 8

    def idx_issue(b_clamped, par):
        row0 = lax.min(row_base + b_clamped * 8, last_row)
        pltpu.async_copy(src_hbm.at[pl.ds(row0, 8)], sidxs[par], sem_i)
        pltpu.async_copy(dst_hbm.at[pl.ds(row0, 8)], didxs[par], sem_i)

    def idx_drain(par):
        pltpu.make_async_copy(src_hbm.at[pl.ds(row_base, 8)], sidxs[par],
                              sem_i).wait()
        pltpu.make_async_copy(dst_hbm.at[pl.ds(row_base, 8)], didxs[par],
                              sem_i).wait()

    def gather(sidx_v, j):
        return pltpu.async_copy(hs_hbm.at[sidx_v.at[j]], bufs[j % 2],
                                sems[j % 2])

    def drain(sidx_v, j):
        pltpu.make_async_copy(hs_hbm.at[sidx_v.at[j]], bufs[j % 2],
                              sems[j % 2]).wait()

    def scatter(didx_v, j):
        pltpu.sync_copy(bufs[j % 2], acc_sh.at[didx_v.at[j]], add=True)

    def run_groups(sidx_v, didx_v):
        gather(sidx_v, 0)
        for j in range(1, 8):
            gather(sidx_v, j)
            drain(sidx_v, j - 1)
            scatter(didx_v, j - 1)
        drain(sidx_v, 7)
        scatter(didx_v, 7)

    idx_issue(0, 0)

    def superbatch(sb2, carry):
        idx_drain(0)
        idx_issue(2 * sb2 + 1, 1)
        run_groups(sidx_a, didx_a)
        idx_drain(1)
        idx_issue(2 * sb2 + 2, 0)
        run_groups(sidx_b, didx_b)
        return carry

    lax.fori_loop(0, n_batches // 2, superbatch, 0)
    idx_drain(0)
    plsc.subcore_barrier()
    for z in range(ROWS_PER_TILE // 128):
        pltpu.sync_copy(acc_sh.at[pl.ds(r0 + z * 128, 128)], buf_a)
        pltpu.sync_copy(buf_a, out_hbm.at[c, pl.ds(r0 + z * 128, 128)])


def _h_body(x_ref, w_ref, h_ref):
    parts = [jnp.dot(x_ref[t], w_ref[...], preferred_element_type=jnp.float32)
             for t in range(T)]
    h_ref[...] = jnp.concatenate(parts, axis=1)


def _scale_body(h_ref, degs_ref, hs_ref, dis_ref):
    i = pl.program_id(0)
    deg = degs_ref[0, :, 0:1] + degs_ref[1, :, 0:1] + 1.0
    dis = lax.rsqrt(deg)
    rows = i * NB + lax.broadcasted_iota(jnp.int32, (NB, 1), 0)
    valid = rows < N
    hs_ref[...] = jnp.where(valid, h_ref[...] * dis, 0.0)
    dis_ref[...] = jnp.where(valid, jnp.broadcast_to(dis, (NB, H)), 0.0)


def _lstm_body(acc_ref, hs_ref, dis_ref, bias_ref, wih_ref, whh_ref,
               b_ref, fcw_ref, fcb_ref, out_ref):
    dis = dis_ref[...][:, 0:1]
    agg = dis * (acc_ref[0] + acc_ref[1] + hs_ref[...])
    g = jnp.maximum(agg + bias_ref[...], 0.0)
    h = jnp.zeros((NB, H), jnp.float32)
    cst = jnp.zeros((NB, H), jnp.float32)
    b = b_ref[...]
    for t in range(T):
        xt = g[:, t * H:(t + 1) * H]
        gates = (jnp.dot(xt, wih_ref[...], preferred_element_type=jnp.float32)
                 + jnp.dot(h, whh_ref[...], preferred_element_type=jnp.float32)
                 + b)
        ii = jax.nn.sigmoid(gates[:, 0:H])
        ff = jax.nn.sigmoid(gates[:, H:2 * H])
        gg = jnp.tanh(gates[:, 2 * H:3 * H])
        oo = jax.nn.sigmoid(gates[:, 3 * H:4 * H])
        cst = ff * cst + ii * gg
        h = oo * jnp.tanh(cst)
    out_ref[...] = (jnp.dot(h, fcw_ref[...], preferred_element_type=jnp.float32)
                    + fcb_ref[...])


def kernel(x_list, edge_index, gcn_weight, gcn_bias, W_ih, W_hh, b_ih, b_hh,
           fc_w, fc_b):
    e = edge_index.shape[1]
    ept = KA * KB * 128
    e_pad = ((e + NW * ept - 1) // (NW * ept)) * (NW * ept)
    n_chunks_a = e_pad // (NW * KA * 128)
    n_batches = e_pad // (NW * 8 * 128)

    pad = N + (jnp.arange(e_pad - e, dtype=jnp.int32) % (NPAD - N))
    src2d = jnp.concatenate([edge_index[0], pad]).reshape(-1, 128)
    dst2d = jnp.concatenate([edge_index[1], pad]).reshape(-1, 128)

    ones_deg = jnp.ones((128, DEGW), jnp.float32)
    zeros_deg = jnp.zeros((NPAD, DEGW), jnp.float32)
    zerosF = jnp.zeros((NPAD, F), jnp.float32)

    deg_k = pl.kernel(
        functools.partial(_deg_body, n_chunks_a),
        out_type=jax.ShapeDtypeStruct((2, NPAD, DEGW), jnp.float32),
        mesh=plsc.VectorSubcoreMesh(**_MESH),
        scratch_types=[
            pltpu.VMEM((KA, 128), jnp.int32),
            pltpu.VMEM((128, DEGW), jnp.float32),
            pltpu.VMEM_SHARED((NPAD, DEGW), jnp.float32),
            pltpu.SemaphoreType.DMA,
            pltpu.SemaphoreType.DMA,
        ],
    )
    degs = deg_k(dst2d, ones_deg, zeros_deg)

    grid = NPAD // NB
    h_all = pl.pallas_call(
        _h_body,
        grid=(grid,),
        in_specs=[
            pl.BlockSpec((T, NB, D), lambda i: (0, i, 0)),
            pl.BlockSpec((D, H), lambda i: (0, 0)),
        ],
        out_specs=pl.BlockSpec((NB, F), lambda i: (i, 0)),
        out_shape=jax.ShapeDtypeStruct((NPAD, F), jnp.float32),
    )(x_list, gcn_weight)
    hs, dis16 = pl.pallas_call(
        _scale_body,
        grid=(grid,),
        in_specs=[
            pl.BlockSpec((NB, F), lambda i: (i, 0)),
            pl.BlockSpec((2, NB, DEGW), lambda i: (0, i, 0)),
        ],
        out_specs=[
            pl.BlockSpec((NB, F), lambda i: (i, 0)),
            pl.BlockSpec((NB, H), lambda i: (i, 0)),
        ],
        out_shape=[
            jax.ShapeDtypeStruct((NPAD, F), jnp.float32),
            jax.ShapeDtypeStruct((NPAD, H), jnp.float32),
        ],
    )(h_all, degs)

    agg_k = pl.kernel(
        functools.partial(_agg_body, n_batches),
        out_type=jax.ShapeDtypeStruct((2, NPAD, F), jnp.float32),
        mesh=plsc.VectorSubcoreMesh(**_MESH),
        scratch_types=[
            pltpu.VMEM((8, 128), jnp.int32),
            pltpu.VMEM((8, 128), jnp.int32),
            pltpu.VMEM((8, 128), jnp.int32),
            pltpu.VMEM((8, 128), jnp.int32),
            pltpu.VMEM((128, F), jnp.float32),
            pltpu.VMEM((128, F), jnp.float32),
            pltpu.VMEM_SHARED((NPAD, F), jnp.float32),
            pltpu.SemaphoreType.DMA,
            pltpu.SemaphoreType.DMA,
            pltpu.SemaphoreType.DMA,
        ],
    )
    acc = agg_k(hs, src2d, dst2d, zerosF)

    bias128 = jnp.tile(gcn_bias, T).reshape(1, F)
    b_all = (b_ih + b_hh).reshape(1, 4 * H)
    out = pl.pallas_call(
        _lstm_body,
        grid=(grid,),
        in_specs=[
            pl.BlockSpec((2, NB, F), lambda i: (0, i, 0)),
            pl.BlockSpec((NB, F), lambda i: (i, 0)),
            pl.BlockSpec((NB, H), lambda i: (i, 0)),
            pl.BlockSpec((1, F), lambda i: (0, 0)),
            pl.BlockSpec((H, 4 * H), lambda i: (0, 0)),
            pl.BlockSpec((H, 4 * H), lambda i: (0, 0)),
            pl.BlockSpec((1, 4 * H), lambda i: (0, 0)),
            pl.BlockSpec((H, 1), lambda i: (0, 0)),
            pl.BlockSpec((1, 1), lambda i: (0, 0)),
        ],
        out_specs=pl.BlockSpec((NB, 1), lambda i: (i, 0)),
        out_shape=jax.ShapeDtypeStruct((NPAD, 1), jnp.float32),
    )(acc, hs, dis16, bias128, W_ih.T, W_hh.T, b_all, fc_w.T,
      fc_b.reshape(1, 1))

    return out[:N, 0]
